# baseline (speedup 1.0000x reference)
import functools

import jax
import jax.numpy as jnp
from jax import lax
from jax.experimental import pallas as pl
from jax.experimental.pallas import tpu as pltpu
from jax.experimental.pallas import tpu_sc as plsc

NC = 2
NS = 16
NW = NC * NS
L = 16

BB = 128
TW = BB + 1
DEPTH = 4


CH = 512


@functools.partial(jax.jit, static_argnums=(1, 2))
def _relayout_call(tT, V, D):
    grid = (V + CH - 1) // CH

    def body(tT_ref, o_ref):
        blk = tT_ref[...]
        t = jnp.transpose(blk)
        a = t.reshape(CH // 2, 2, D)
        o_ref[:, 0:D] = a[:, 0, :]
        o_ref[:, D:2 * D] = a[:, 1, :]

    return pl.pallas_call(
        body,
        grid=(grid,),
        in_specs=[pl.BlockSpec((D, CH), lambda i: (0, i))],
        out_specs=pl.BlockSpec((CH // 2, 2 * D), lambda i: (i, 0)),
        out_shape=jax.ShapeDtypeStruct((V // 2, 2 * D), jnp.float32),
    )(tT)


@functools.partial(jax.jit, static_argnums=(3, 4, 5))
def _emb_call(xT, token_table, pos, B, S, D):
    mesh = plsc.VectorSubcoreMesh(core_axis_name="c", subcore_axis_name="s")
    n_bblk = B // BB
    assert n_bblk == NW and S % DEPTH == 0

    @functools.partial(
        pl.kernel,
        out_type=jax.ShapeDtypeStruct((S, D // 8, n_bblk, 8, BB), jnp.float32),
        mesh=mesh,
        scratch_types=[
            pltpu.VMEM((S, D), jnp.float32),
            pltpu.VMEM((S, BB), jnp.int32),
            pltpu.VMEM((DEPTH, BB, D), jnp.float32),
            pltpu.VMEM((DEPTH, D // 8, 8, TW), jnp.float32),
            pltpu.SemaphoreType.DMA((DEPTH,)),
            pltpu.SemaphoreType.DMA((DEPTH,)),
        ],
        compiler_params=pltpu.CompilerParams(
            use_tc_tiling_on_sc=False, needs_layout_passes=False),
    )
    def emb(xT_hbm, tok_hbm, pos_hbm, out_hbm, pos_v, idx_v, g_v, t_v,
            sem_g, sem_o):
        wid = lax.axis_index("c") * NS + lax.axis_index("s")

        pltpu.sync_copy(pos_hbm, pos_v)
        pltpu.sync_copy(xT_hbm.at[:, pl.ds(wid * BB, BB)], idx_v)

        def start_gather(s, buf):
            pltpu.async_copy(tok_hbm.at[idx_v.at[s]], g_v.at[buf],
                             sem_g.at[buf])

        def wait_gather(s, buf):
            pltpu.make_async_copy(tok_hbm.at[idx_v.at[s]], g_v.at[buf],
                                  sem_g.at[buf]).wait()

        def start_out(s, buf):
            pltpu.async_copy(t_v.at[buf, :, :, pl.ds(0, BB)],
                             out_hbm.at[s, :, wid], sem_o.at[buf])

        def wait_out(s, buf):
            pltpu.make_async_copy(t_v.at[buf, :, :, pl.ds(0, BB)],
                                  out_hbm.at[s, :, wid], sem_o.at[buf]).wait()

        lane = jnp.arange(L, dtype=jnp.int32)
        dtvecs = [(lane + L * i) // 8 for i in range(D // L)]
        divecs = [(lane + L * i) % 8 for i in range(D // L)]
        bufids = [jnp.full((L,), b, dtype=jnp.int32) for b in range(DEPTH)]

        def transpose_add(s, buf):
            pv = [pos_v[s, pl.ds(i * L, L)] for i in range(D // L)]

            @plsc.parallel_loop(0, BB, unroll=8)
            def body(bi):
                bisplat = jnp.broadcast_to(bi, (L,)).astype(jnp.int32)
                for i in range(D // L):
                    v = g_v[buf, bi, pl.ds(i * L, L)] + pv[i]
                    plsc.store_scatter(
                        t_v, [bufids[buf], dtvecs[i], divecs[i], bisplat], v)

        def chunk_step(s, buf, k):
            wait_gather(s, buf)

            @pl.when(s + 2 < S)
            def _():
                start_gather(s + 2, (buf + 2) % DEPTH)

            @pl.when(k > 0)
            def _():
                wait_out(s - DEPTH, buf)

            transpose_add(s, buf)
            start_out(s, buf)

        start_gather(0, 0)
        start_gather(1, 1)

        def loop_body(k, carry):
            for j in range(DEPTH):
                chunk_step(DEPTH * k + j, j, k)
            return carry

        lax.fori_loop(0, S // DEPTH, loop_body, 0)
        for j in range(DEPTH):
            wait_out(S - DEPTH + j, j)

    return emb(xT, token_table, pos)


def kernel(x, token_table, position_table):
    B, S = x.shape
    D = token_table.shape[1]
    xT = x.T.astype(jnp.int32)
    pos = position_table[:S]
    V = token_table.shape[0]
    tok2 = _relayout_call(token_table.T, V, D).reshape(V, D)
    out5 = _emb_call(xT, tok2, pos, B, S, D)
    out = out5.transpose(2, 4, 0, 1, 3).reshape(B, S, D)
    return out


# speedup vs baseline: 1.7480x; 1.7480x over previous
import functools

import jax
import jax.numpy as jnp
from jax import lax
from jax.experimental import pallas as pl
from jax.experimental.pallas import tpu as pltpu
from jax.experimental.pallas import tpu_sc as plsc

NC = 2
NS = 16
NW = NC * NS
L = 16

BB = 128
TW = BB + 1
DEPTH = 4


RCH = 224
NCH = 144


@functools.partial(jax.jit, static_argnums=(1, 2))
def _strip_call(tok, V, D):
    mesh = plsc.VectorSubcoreMesh(core_axis_name="c", subcore_axis_name="s")
    last = V - RCH

    @functools.partial(
        pl.kernel,
        out_type=jax.ShapeDtypeStruct((V // 2, 2 * D), jnp.float32),
        mesh=mesh,
        scratch_types=[
            pltpu.VMEM((2, RCH, D), jnp.float32),
            pltpu.VMEM((2, RCH // 2, 2 * D), jnp.float32),
            pltpu.SemaphoreType.DMA((2,)),
            pltpu.SemaphoreType.DMA((2,)),
        ],
        compiler_params=pltpu.CompilerParams(use_tc_tiling_on_sc=True),
    )
    def strip(tok_hbm, out_hbm, va, vb, sem_rd, sem_wr):
        wid = lax.axis_index("c") * NS + lax.axis_index("s")

        def base_of(c):
            return pl.multiple_of(
                jnp.minimum((wid * NCH + c) * RCH, last), 16)

        def start_rd(c, buf):
            pltpu.async_copy(tok_hbm.at[pl.ds(base_of(c), RCH), :],
                             va.at[buf], sem_rd.at[buf])

        def wait_rd(c, buf):
            pltpu.make_async_copy(tok_hbm.at[pl.ds(base_of(c), RCH), :],
                                  va.at[buf], sem_rd.at[buf]).wait()

        def start_wr(c, buf):
            pltpu.async_copy(
                vb.at[buf],
                out_hbm.at[pl.ds(pl.multiple_of(base_of(c) // 2, 8),
                                 RCH // 2), :],
                sem_wr.at[buf])

        def wait_wr(c, buf):
            pltpu.make_async_copy(
                vb.at[buf],
                out_hbm.at[pl.ds(pl.multiple_of(base_of(c) // 2, 8),
                                 RCH // 2), :],
                sem_wr.at[buf]).wait()

        def merge(buf):
            @plsc.parallel_loop(0, RCH // 2, unroll=8)
            def body(q):
                for h in range(2):
                    for i in range(D // L):
                        vb[buf, q, pl.ds(h * D + i * L, L)] = (
                            va[buf, 2 * q + h, pl.ds(i * L, L)])

        start_rd(0, 0)

        def body(k, carry):
            for j in range(2):
                c = 2 * k + j
                buf = j
                wait_rd(c, buf)

                @pl.when(c + 1 < NCH)
                def _():
                    start_rd(c + 1, 1 - buf)

                @pl.when(c >= 2)
                def _():
                    wait_wr(c - 2, buf)

                merge(buf)
                start_wr(c, buf)
            return carry

        lax.fori_loop(0, NCH // 2, body, 0)
        wait_wr(NCH - 2, 0)
        wait_wr(NCH - 1, 1)

    return strip(tok)


@functools.partial(jax.jit, static_argnums=(3, 4, 5))
def _emb_call(xT, token_table, pos, B, S, D):
    mesh = plsc.VectorSubcoreMesh(core_axis_name="c", subcore_axis_name="s")
    n_bblk = B // BB
    assert n_bblk == NW and S % DEPTH == 0

    @functools.partial(
        pl.kernel,
        out_type=jax.ShapeDtypeStruct((S, D // 8, n_bblk, 8, BB), jnp.float32),
        mesh=mesh,
        scratch_types=[
            pltpu.VMEM((S, D), jnp.float32),
            pltpu.VMEM((S, BB), jnp.int32),
            pltpu.VMEM((DEPTH, BB, D), jnp.float32),
            pltpu.VMEM((DEPTH, D // 8, 8, TW), jnp.float32),
            pltpu.SemaphoreType.DMA((DEPTH,)),
            pltpu.SemaphoreType.DMA((DEPTH,)),
        ],
        compiler_params=pltpu.CompilerParams(
            use_tc_tiling_on_sc=False, needs_layout_passes=False),
    )
    def emb(xT_hbm, tok_hbm, pos_hbm, out_hbm, pos_v, idx_v, g_v, t_v,
            sem_g, sem_o):
        wid = lax.axis_index("c") * NS + lax.axis_index("s")

        pltpu.sync_copy(pos_hbm, pos_v)
        pltpu.sync_copy(xT_hbm.at[:, pl.ds(wid * BB, BB)], idx_v)

        def start_gather(s, buf):
            pltpu.async_copy(tok_hbm.at[idx_v.at[s]], g_v.at[buf],
                             sem_g.at[buf])

        def wait_gather(s, buf):
            pltpu.make_async_copy(tok_hbm.at[idx_v.at[s]], g_v.at[buf],
                                  sem_g.at[buf]).wait()

        def start_out(s, buf):
            pltpu.async_copy(t_v.at[buf, :, :, pl.ds(0, BB)],
                             out_hbm.at[s, :, wid], sem_o.at[buf])

        def wait_out(s, buf):
            pltpu.make_async_copy(t_v.at[buf, :, :, pl.ds(0, BB)],
                                  out_hbm.at[s, :, wid], sem_o.at[buf]).wait()

        lane = jnp.arange(L, dtype=jnp.int32)
        dtvecs = [(lane + L * i) // 8 for i in range(D // L)]
        divecs = [(lane + L * i) % 8 for i in range(D // L)]
        bufids = [jnp.full((L,), b, dtype=jnp.int32) for b in range(DEPTH)]

        def transpose_add(s, buf):
            pv = [pos_v[s, pl.ds(i * L, L)] for i in range(D // L)]

            @plsc.parallel_loop(0, BB, unroll=8)
            def body(bi):
                bisplat = jnp.broadcast_to(bi, (L,)).astype(jnp.int32)
                for i in range(D // L):
                    v = g_v[buf, bi, pl.ds(i * L, L)] + pv[i]
                    plsc.store_scatter(
                        t_v, [bufids[buf], dtvecs[i], divecs[i], bisplat], v)

        def chunk_step(s, buf, k):
            wait_gather(s, buf)

            @pl.when(s + 2 < S)
            def _():
                start_gather(s + 2, (buf + 2) % DEPTH)

            @pl.when(k > 0)
            def _():
                wait_out(s - DEPTH, buf)

            transpose_add(s, buf)
            start_out(s, buf)

        start_gather(0, 0)
        start_gather(1, 1)

        def loop_body(k, carry):
            for j in range(DEPTH):
                chunk_step(DEPTH * k + j, j, k)
            return carry

        lax.fori_loop(0, S // DEPTH, loop_body, 0)
        for j in range(DEPTH):
            wait_out(S - DEPTH + j, j)

    return emb(xT, token_table, pos)


def kernel(x, token_table, position_table):
    B, S = x.shape
    D = token_table.shape[1]
    xT = x.T.astype(jnp.int32)
    pos = position_table[:S]
    V = token_table.shape[0]
    tok2 = _strip_call(token_table, V, D).reshape(V, D)
    out5 = _emb_call(xT, tok2, pos, B, S, D)
    out = out5.transpose(2, 4, 0, 1, 3).reshape(B, S, D)
    return out


# speedup vs baseline: 1.8663x; 1.0677x over previous
import functools

import jax
import jax.numpy as jnp
from jax import lax
from jax.experimental import pallas as pl
from jax.experimental.pallas import tpu as pltpu
from jax.experimental.pallas import tpu_sc as plsc

NC = 2
NS = 16
NW = NC * NS
L = 16

BB = 128
TW = BB + 1
DEPTH = 4


@functools.partial(jax.jit, static_argnums=(3, 4, 5))
def _emb_call(xT, token_table, pos, B, S, D):
    mesh = plsc.VectorSubcoreMesh(core_axis_name="c", subcore_axis_name="s")
    n_bblk = B // BB
    assert n_bblk == NW and S % DEPTH == 0

    @functools.partial(
        pl.kernel,
        out_type=jax.ShapeDtypeStruct((S, D // 8, n_bblk, 8, BB), jnp.float32),
        mesh=mesh,
        scratch_types=[
            pltpu.VMEM((S, D), jnp.float32),
            pltpu.VMEM((S, BB), jnp.int32),
            pltpu.VMEM((DEPTH, BB, D), jnp.float32),
            pltpu.VMEM((DEPTH, D // 8, 8, TW), jnp.float32),
            pltpu.SemaphoreType.DMA((DEPTH,)),
            pltpu.SemaphoreType.DMA((DEPTH,)),
        ],
        compiler_params=pltpu.CompilerParams(
            use_tc_tiling_on_sc=False, needs_layout_passes=False),
    )
    def emb(xT_hbm, tok_hbm, pos_hbm, out_hbm, pos_v, idx_v, g_v, t_v,
            sem_g, sem_o):
        wid = lax.axis_index("c") * NS + lax.axis_index("s")

        pltpu.sync_copy(pos_hbm, pos_v)
        pltpu.sync_copy(xT_hbm.at[:, pl.ds(wid * BB, BB)], idx_v)

        def start_gather(s, buf):
            pltpu.async_copy(tok_hbm.at[idx_v.at[s]], g_v.at[buf],
                             sem_g.at[buf])

        def wait_gather(s, buf):
            pltpu.make_async_copy(tok_hbm.at[idx_v.at[s]], g_v.at[buf],
                                  sem_g.at[buf]).wait()

        def start_out(s, buf):
            pltpu.async_copy(t_v.at[buf, :, :, pl.ds(0, BB)],
                             out_hbm.at[s, :, wid], sem_o.at[buf])

        def wait_out(s, buf):
            pltpu.make_async_copy(t_v.at[buf, :, :, pl.ds(0, BB)],
                                  out_hbm.at[s, :, wid], sem_o.at[buf]).wait()

        lane = jnp.arange(L, dtype=jnp.int32)
        dtvecs = [(lane + L * i) // 8 for i in range(D // L)]
        divecs = [(lane + L * i) % 8 for i in range(D // L)]
        bufids = [jnp.full((L,), b, dtype=jnp.int32) for b in range(DEPTH)]

        def transpose_add(s, buf):
            pv = [pos_v[s, pl.ds(i * L, L)] for i in range(D // L)]

            @plsc.parallel_loop(0, BB, unroll=8)
            def body(bi):
                bisplat = jnp.broadcast_to(bi, (L,)).astype(jnp.int32)
                for i in range(D // L):
                    v = g_v[buf, bi, pl.ds(i * L, L)] + pv[i]
                    plsc.store_scatter(
                        t_v, [bufids[buf], dtvecs[i], divecs[i], bisplat], v)

        def chunk_step(s, buf, k):
            wait_gather(s, buf)

            @pl.when(s + 2 < S)
            def _():
                start_gather(s + 2, (buf + 2) % DEPTH)

            @pl.when(k > 0)
            def _():
                wait_out(s - DEPTH, buf)

            transpose_add(s, buf)
            start_out(s, buf)

        start_gather(0, 0)
        start_gather(1, 1)

        def loop_body(k, carry):
            for j in range(DEPTH):
                chunk_step(DEPTH * k + j, j, k)
            return carry

        lax.fori_loop(0, S // DEPTH, loop_body, 0)
        for j in range(DEPTH):
            wait_out(S - DEPTH + j, j)

    return emb(xT, token_table, pos)


def kernel(x, token_table, position_table):
    B, S = x.shape
    D = token_table.shape[1]
    xT = x.T.astype(jnp.int32)
    pos = position_table[:S]
    out5 = _emb_call(xT, token_table, pos, B, S, D)
    out = out5.transpose(2, 4, 0, 1, 3).reshape(B, S, D)
    return out
